# Initial kernel scaffold; baseline (speedup 1.0000x reference)
import jax
import jax.numpy as jnp
from jax.experimental import pallas as pl


def kernel(x, edge_index, Wl1, bl1, Wr1, br1, att1, bias1, Wl2, bl2, Wr2, br2, att2, bias2):
    raise NotImplementedError("write your pallas kernel here")


# speedup vs baseline: 3.7829x; 3.7829x over previous
import functools

import jax
import jax.numpy as jnp
from jax import lax
from jax.experimental import pallas as pl
from jax.experimental.pallas import tpu as pltpu
from jax.experimental.pallas import tpu_sc as plsc

NS = 16
NC = 2
NW = NC * NS
BK = 16
SLOPE = 0.2
EPS = 1e-16

_F32 = jnp.float32
_I32 = jnp.int32

_SC_PARAMS = pltpu.CompilerParams(use_tc_tiling_on_sc=False,
                                  needs_layout_passes=False)


def _lanes():
    return lax.iota(_I32, 16)


def _splat_i(v):
    return jnp.zeros((16,), _I32) + v


def _zero16():
    return jnp.zeros((16,), _F32)


def _tc1_body(x_ref, wl_ref, bl_ref, wr_ref, br_ref, xl_ref, xr_ref, xlc_ref):
    x = x_ref[...]
    xl = jnp.dot(x, wl_ref[...], preferred_element_type=_F32,
                 precision=lax.Precision.HIGHEST) + bl_ref[...]
    xr = jnp.dot(x, wr_ref[...], preferred_element_type=_F32,
                 precision=lax.Precision.HIGHEST) + br_ref[...]
    xl_ref[...] = xl
    xr_ref[...] = xr
    for c in range(3):
        xlc_ref[c] = xl[:, (c + 1) * 128:(c + 2) * 128]


def _tc1(x, wl, bl, wr, br, N, D, HF):
    BN = 1000
    return pl.pallas_call(
        _tc1_body,
        grid=(N // BN,),
        in_specs=[
            pl.BlockSpec((BN, D), lambda i: (i, 0)),
            pl.BlockSpec((D, HF), lambda i: (0, 0)),
            pl.BlockSpec((1, HF), lambda i: (0, 0)),
            pl.BlockSpec((D, HF), lambda i: (0, 0)),
            pl.BlockSpec((1, HF), lambda i: (0, 0)),
        ],
        out_specs=[
            pl.BlockSpec((BN, HF), lambda i: (i, 0)),
            pl.BlockSpec((BN, HF), lambda i: (i, 0)),
            pl.BlockSpec((3, BN, 128), lambda i: (0, i, 0)),
        ],
        out_shape=[
            jax.ShapeDtypeStruct((N, HF), _F32),
            jax.ShapeDtypeStruct((N, HF), _F32),
            jax.ShapeDtypeStruct((3, N, 128), _F32),
        ],
    )(x, wl, bl, wr, br)


def _sc1_body(N, E, xl_hbm, xr_hbm, xlc_hbm, src_hbm, dst_hbm, att_hbm,
              un_hbm, den_hbm, exq_hbm,
              accum, src_buf, dst_buf, xl_rows, xr_rows, xc_rows,
              ex_stage, row_buf, att_buf, zero_buf, sem1, sem2):
    c = lax.axis_index("c")
    s = lax.axis_index("s")
    wid = c * NS + s
    EW = E // NW
    NB = EW // BK
    RS = N // NS
    ZR = 125
    lanes = _lanes()

    pltpu.sync_copy(att_hbm, att_buf)

    @pl.loop(0, ZR)
    def _(i):
        for j in range(144 // 16):
            zero_buf[i, pl.ds(j * 16, 16)] = _zero16()

    @pl.loop(0, BK)
    def _(e):
        row_buf[e, pl.ds(128, 16)] = _zero16()
        ex_stage[e, pl.ds(0, 16)] = _zero16()

    def zero_accum():
        @pl.loop(0, RS // ZR)
        def _(k):
            pltpu.sync_copy(zero_buf, accum.at[pl.ds(s * RS + k * ZR, ZR), :])

    def readout(chunk):
        pltpu.sync_copy(accum.at[pl.ds(s * RS, RS), pl.ds(0, 128)],
                        un_hbm.at[c, chunk, pl.ds(s * RS, RS)])
        if chunk == 0:
            pltpu.sync_copy(accum.at[pl.ds(s * RS, RS), pl.ds(128, 16)],
                            den_hbm.at[c, pl.ds(s * RS, RS)])

    zero_accum()
    plsc.subcore_barrier()

    @pl.loop(0, NB)
    def _(b):
        base = wid * EW + b * BK
        pltpu.sync_copy(src_hbm.at[pl.ds(base, BK)], src_buf)
        pltpu.sync_copy(dst_hbm.at[pl.ds(base, BK)], dst_buf)
        cp1 = pltpu.async_copy(xl_hbm.at[src_buf], xl_rows, sem1)
        cp2 = pltpu.async_copy(xr_hbm.at[dst_buf], xr_rows, sem2)
        cp1.wait()
        cp2.wait()
        for h in range(8):
            def fbody(jo, acc, h=h):
                attv = att_buf[h, pl.ds(jo * 16, 16)]
                for jj in range(16):
                    col = _splat_i(h * 64 + jo * 16 + jj)
                    xlv = plsc.load_gather(xl_rows, [lanes, col])
                    xrv = plsc.load_gather(xr_rows, [lanes, col])
                    m = xlv + xrv
                    lk = jnp.maximum(m, SLOPE * m)
                    acc = acc + attv[jj] * lk
                return acc
            acc = lax.fori_loop(0, 4, fbody, _zero16())
            exh = jnp.exp(acc)
            plsc.store_scatter(ex_stage, [lanes, _splat_i(h)], exh)
            plsc.store_scatter(row_buf, [lanes, _splat_i(128 + h)], exh)

        @pl.loop(0, BK)
        def _(e):
            exv = ex_stage[e, pl.ds(0, 16)]
            for j in range(8):
                row_buf[e, pl.ds(j * 16, 16)] = (
                    xl_rows[e, pl.ds(j * 16, 16)] * exv[j // 4])

        pltpu.sync_copy(ex_stage, exq_hbm.at[pl.ds(base, BK)])
        pltpu.sync_copy(row_buf, accum.at[dst_buf], add=True)

    plsc.subcore_barrier()
    readout(0)
    zero_accum()
    plsc.subcore_barrier()

    for ch in range(1, 4):
        @pl.loop(0, BK)
        def _(e):
            row_buf[e, pl.ds(128, 16)] = _zero16()

        @pl.loop(0, NB)
        def _(b, ch=ch):
            base = wid * EW + b * BK
            pltpu.sync_copy(src_hbm.at[pl.ds(base, BK)], src_buf)
            pltpu.sync_copy(dst_hbm.at[pl.ds(base, BK)], dst_buf)
            cp1 = pltpu.async_copy(xlc_hbm.at[ch - 1].at[src_buf], xc_rows, sem1)
            cp1.wait()
            pltpu.sync_copy(exq_hbm.at[pl.ds(base, BK)], ex_stage)

            @pl.loop(0, BK)
            def _(e, ch=ch):
                exv = ex_stage[e, pl.ds(0, 16)]
                for j in range(8):
                    row_buf[e, pl.ds(j * 16, 16)] = (
                        xc_rows[e, pl.ds(j * 16, 16)] * exv[2 * ch + j // 4])

            pltpu.sync_copy(row_buf, accum.at[dst_buf], add=True)

        plsc.subcore_barrier()
        readout(ch)
        if ch < 3:
            zero_accum()
        plsc.subcore_barrier()


def _sc1(xl, xr, xlc, src, dst, att, N, E):
    mesh = plsc.VectorSubcoreMesh(core_axis_name="c", subcore_axis_name="s")
    return pl.kernel(
        functools.partial(_sc1_body, N, E),
        out_type=(
            jax.ShapeDtypeStruct((NC, 4, N, 128), _F32),
            jax.ShapeDtypeStruct((NC, N, 16), _F32),
            jax.ShapeDtypeStruct((E, 16), _F32),
        ),
        mesh=mesh,
        compiler_params=_SC_PARAMS,
        scratch_types=[
            pltpu.VMEM_SHARED((N, 144), _F32),
            pltpu.VMEM((BK,), _I32),
            pltpu.VMEM((BK,), _I32),
            pltpu.VMEM((BK, 512), _F32),
            pltpu.VMEM((BK, 512), _F32),
            pltpu.VMEM((BK, 128), _F32),
            pltpu.VMEM((BK, 16), _F32),
            pltpu.VMEM((BK, 144), _F32),
            pltpu.VMEM((8, 64), _F32),
            pltpu.VMEM((125, 144), _F32),
            pltpu.SemaphoreType.DMA,
            pltpu.SemaphoreType.DMA,
        ],
    )(xl, xr, xlc, src, dst, att)


def _tc2_body(u_ref, d_ref, b1_ref, wl2_ref, bl2_ref, wr2_ref, br2_ref,
              xl2_ref, xr2_ref):
    u = u_ref[...]
    us = u[0] + u[1]
    d = d_ref[...]
    dsum = d[0] + d[1]
    cols = []
    for ch in range(4):
        for k in range(2):
            dh = dsum[:, 2 * ch + k:2 * ch + k + 1] + EPS
            cols.append(us[ch][:, 64 * k:64 * (k + 1)] / dh)
    h = jnp.concatenate(cols, axis=1) + b1_ref[...]
    h = jnp.maximum(h, 0.0)
    xl2_ref[...] = jnp.dot(h, wl2_ref[...], preferred_element_type=_F32,
                           precision=lax.Precision.HIGHEST) + bl2_ref[...]
    xr2_ref[...] = jnp.dot(h, wr2_ref[...], preferred_element_type=_F32,
                           precision=lax.Precision.HIGHEST) + br2_ref[...]


def _tc2(un, den, b1, wl2, bl2, wr2, br2, N, HF):
    BN = 1000
    return pl.pallas_call(
        _tc2_body,
        grid=(N // BN,),
        in_specs=[
            pl.BlockSpec((NC, 4, BN, 128), lambda i: (0, 0, i, 0)),
            pl.BlockSpec((NC, BN, 16), lambda i: (0, i, 0)),
            pl.BlockSpec((1, HF), lambda i: (0, 0)),
            pl.BlockSpec((HF, 1), lambda i: (0, 0)),
            pl.BlockSpec((1, 1), lambda i: (0, 0)),
            pl.BlockSpec((HF, 1), lambda i: (0, 0)),
            pl.BlockSpec((1, 1), lambda i: (0, 0)),
        ],
        out_specs=[
            pl.BlockSpec((BN, 1), lambda i: (i, 0)),
            pl.BlockSpec((BN, 1), lambda i: (i, 0)),
        ],
        out_shape=[
            jax.ShapeDtypeStruct((N, 1), _F32),
            jax.ShapeDtypeStruct((N, 1), _F32),
        ],
    )(un, den, b1, wl2, bl2, wr2, br2)


def _sc2a_body(N, E, xl2_hbm, xr2_hbm, src_hbm, dst_hbm, att2_hbm,
               parts_hbm, ex2q_hbm,
               accum2, xl2_buf, xr2_buf, att2_buf, src_buf, dst_buf,
               ex2_buf, row2_buf, zero2_buf):
    c = lax.axis_index("c")
    s = lax.axis_index("s")
    wid = c * NS + s
    EW = E // NW
    NB = EW // BK
    RS = N // NS
    ZR = 125
    lanes = _lanes()
    zi = _splat_i(0)

    pltpu.sync_copy(xl2_hbm, xl2_buf)
    pltpu.sync_copy(xr2_hbm, xr2_buf)
    pltpu.sync_copy(att2_hbm, att2_buf)

    @pl.loop(0, ZR)
    def _(i):
        zero2_buf[i, pl.ds(0, 16)] = _zero16()

    @pl.loop(0, BK)
    def _(e):
        row2_buf[e, pl.ds(0, 16)] = _zero16()

    @pl.loop(0, RS // ZR)
    def _(k):
        pltpu.sync_copy(zero2_buf, accum2.at[pl.ds(s * RS + k * ZR, ZR), :])

    plsc.subcore_barrier()
    attv = att2_buf[...]

    @pl.loop(0, NB)
    def _(b):
        base = wid * EW + b * BK
        pltpu.sync_copy(src_hbm.at[pl.ds(base, BK)], src_buf)
        pltpu.sync_copy(dst_hbm.at[pl.ds(base, BK)], dst_buf)
        srcv = src_buf[...]
        dstv = dst_buf[...]
        xls = plsc.load_gather(xl2_buf, [srcv])
        xrd = plsc.load_gather(xr2_buf, [dstv])
        m = xls + xrd
        lk = jnp.maximum(m, SLOPE * m)
        ex2 = jnp.exp(attv * lk)
        plsc.store_scatter(row2_buf, [lanes, _splat_i(0)], ex2 * xls)
        plsc.store_scatter(row2_buf, [lanes, _splat_i(1)], ex2)
        ex2_buf[...] = ex2
        pltpu.sync_copy(ex2_buf, ex2q_hbm.at[pl.ds(base, BK)])
        pltpu.sync_copy(row2_buf, accum2.at[dst_buf], add=True)

    plsc.subcore_barrier()
    pltpu.sync_copy(accum2.at[pl.ds(s * RS, RS), :],
                    parts_hbm.at[c, pl.ds(s * RS, RS)])


def _sc2a(xl2, xr2, src, dst, att2v, N, E):
    mesh = plsc.VectorSubcoreMesh(core_axis_name="c", subcore_axis_name="s")
    return pl.kernel(
        functools.partial(_sc2a_body, N, E),
        out_type=(
            jax.ShapeDtypeStruct((NC, N, 16), _F32),
            jax.ShapeDtypeStruct((E,), _F32),
        ),
        mesh=mesh,
        compiler_params=_SC_PARAMS,
        scratch_types=[
            pltpu.VMEM_SHARED((N, 16), _F32),
            pltpu.VMEM((N,), _F32),
            pltpu.VMEM((N,), _F32),
            pltpu.VMEM((16,), _F32),
            pltpu.VMEM((BK,), _I32),
            pltpu.VMEM((BK,), _I32),
            pltpu.VMEM((BK,), _F32),
            pltpu.VMEM((BK, 16), _F32),
            pltpu.VMEM((125, 16), _F32),
        ],
    )(xl2, xr2, src, dst, att2v)


def _tc2b_body(p_ref, b2_ref, out2_ref, dtot_ref):
    p = p_ref[...]
    u = p[0, :, 0:1] + p[1, :, 0:1]
    dd = p[0, :, 1:2] + p[1, :, 1:2] + EPS
    out2_ref[...] = u / dd + b2_ref[...]
    dtot_ref[...] = dd


def _tc2b(parts, b2, N):
    BN = 1000
    return pl.pallas_call(
        _tc2b_body,
        grid=(N // BN,),
        in_specs=[
            pl.BlockSpec((NC, BN, 16), lambda i: (0, i, 0)),
            pl.BlockSpec((1, 1), lambda i: (0, 0)),
        ],
        out_specs=[
            pl.BlockSpec((BN, 1), lambda i: (i, 0)),
            pl.BlockSpec((BN, 1), lambda i: (i, 0)),
        ],
        out_shape=[
            jax.ShapeDtypeStruct((N, 1), _F32),
            jax.ShapeDtypeStruct((N, 1), _F32),
        ],
    )(parts, b2)


def _sc2b_body(N, E, dtot_hbm, dst_hbm, ex2q_hbm, a2_hbm,
               dt_buf, dst_buf, e2_buf, a2_buf):
    c = lax.axis_index("c")
    s = lax.axis_index("s")
    wid = c * NS + s
    EW = E // NW
    NB = EW // BK
    zi = _splat_i(0)

    pltpu.sync_copy(dtot_hbm, dt_buf)

    @pl.loop(0, NB)
    def _(b):
        base = wid * EW + b * BK
        pltpu.sync_copy(dst_hbm.at[pl.ds(base, BK)], dst_buf)
        pltpu.sync_copy(ex2q_hbm.at[pl.ds(base, BK)], e2_buf)
        dstv = dst_buf[...]
        dv = plsc.load_gather(dt_buf, [dstv])
        a2_buf[...] = e2_buf[...] / dv
        pltpu.sync_copy(a2_buf, a2_hbm.at[pl.ds(base, BK)])


def _sc2b(dtot, dst, ex2q, N, E):
    mesh = plsc.VectorSubcoreMesh(core_axis_name="c", subcore_axis_name="s")
    return pl.kernel(
        functools.partial(_sc2b_body, N, E),
        out_type=jax.ShapeDtypeStruct((E,), _F32),
        mesh=mesh,
        compiler_params=_SC_PARAMS,
        scratch_types=[
            pltpu.VMEM((N,), _F32),
            pltpu.VMEM((BK,), _I32),
            pltpu.VMEM((BK,), _F32),
            pltpu.VMEM((BK,), _F32),
        ],
    )(dtot, dst, ex2q)


def kernel(x, edge_index, Wl1, bl1, Wr1, br1, att1, bias1,
           Wl2, bl2, Wr2, br2, att2, bias2):
    N, D = x.shape
    E = edge_index.shape[1]
    HF = Wl1.shape[1]
    src = edge_index[0]
    dst = edge_index[1]

    xl, xr, xlc = _tc1(x, Wl1, bl1.reshape(1, HF), Wr1, br1.reshape(1, HF),
                       N, D, HF)
    un, den, exq = _sc1(xl, xr, xlc, src, dst, att1, N, E)
    xl2, xr2 = _tc2(un, den, bias1.reshape(1, HF), Wl2, bl2.reshape(1, 1),
                    Wr2, br2.reshape(1, 1), N, HF)
    att2v = jnp.full((16,), att2[0, 0], _F32)
    parts, ex2q = _sc2a(xl2.reshape(N), xr2.reshape(N), src, dst, att2v, N, E)
    out2, dtot = _tc2b(parts, bias2.reshape(1, 1), N)
    a2 = _sc2b(dtot.reshape(N), dst, ex2q, N, E)
    return out2, edge_index, a2.reshape(E, 1)


# speedup vs baseline: 6.6715x; 1.7636x over previous
import functools

import jax
import jax.numpy as jnp
from jax import lax
from jax.experimental import pallas as pl
from jax.experimental.pallas import tpu as pltpu
from jax.experimental.pallas import tpu_sc as plsc

NS = 16
NC = 2
NW = NC * NS
BK = 16
SLOPE = 0.2
EPS = 1e-16

_F32 = jnp.float32
_I32 = jnp.int32

_SC_PARAMS = pltpu.CompilerParams(use_tc_tiling_on_sc=False,
                                  needs_layout_passes=False)


def _lanes():
    return lax.iota(_I32, 16)


def _splat_i(v):
    return jnp.zeros((16,), _I32) + v


def _zero16():
    return jnp.zeros((16,), _F32)


def _tc1_body(x_ref, wl_ref, bl_ref, wr_ref, br_ref, xl_ref, xr_ref, xlc_ref):
    x = x_ref[...]
    xl = jnp.dot(x, wl_ref[...], preferred_element_type=_F32,
                 precision=lax.Precision.HIGHEST) + bl_ref[...]
    xr = jnp.dot(x, wr_ref[...], preferred_element_type=_F32,
                 precision=lax.Precision.HIGHEST) + br_ref[...]
    xl_ref[...] = xl
    xr_ref[...] = xr
    for c in range(4):
        xlc_ref[c] = xl[:, c * 128:(c + 1) * 128]


def _tc1(x, wl, bl, wr, br, N, D, HF):
    BN = 1000
    return pl.pallas_call(
        _tc1_body,
        grid=(N // BN,),
        in_specs=[
            pl.BlockSpec((BN, D), lambda i: (i, 0)),
            pl.BlockSpec((D, HF), lambda i: (0, 0)),
            pl.BlockSpec((1, HF), lambda i: (0, 0)),
            pl.BlockSpec((D, HF), lambda i: (0, 0)),
            pl.BlockSpec((1, HF), lambda i: (0, 0)),
        ],
        out_specs=[
            pl.BlockSpec((BN, HF), lambda i: (i, 0)),
            pl.BlockSpec((BN, HF), lambda i: (i, 0)),
            pl.BlockSpec((4, BN, 128), lambda i: (0, i, 0)),
        ],
        out_shape=[
            jax.ShapeDtypeStruct((N, HF), _F32),
            jax.ShapeDtypeStruct((N, HF), _F32),
            jax.ShapeDtypeStruct((4, N, 128), _F32),
        ],
    )(x, wl, bl, wr, br)


def _sc1a_body(N, E, xl_hbm, xr_hbm, src_hbm, dst_hbm, att_hbm,
               den_hbm, exq_hbm,
               accum, src_slab, dst_slab, xl_rows2, xr_rows2,
               ex_stage2, att_buf, zero_buf, sem_xl, sem_xr, sem_ex):
    c = lax.axis_index("c")
    s = lax.axis_index("s")
    wid = c * NS + s
    EW = E // NW
    NB = EW // BK
    RS = N // NS
    lanes = _lanes()

    pltpu.sync_copy(att_hbm, att_buf)
    pltpu.sync_copy(src_hbm.at[pl.ds(wid * EW, EW)], src_slab)
    pltpu.sync_copy(dst_hbm.at[pl.ds(wid * EW, EW)], dst_slab)

    @pl.loop(0, 25)
    def _(i):
        zero_buf[i, pl.ds(0, 16)] = _zero16()

    @pl.loop(0, BK)
    def _(e):
        for p in range(2):
            ex_stage2[p, e, pl.ds(0, 16)] = _zero16()

    @pl.loop(0, RS // 25)
    def _(k):
        pltpu.sync_copy(zero_buf, accum.at[pl.ds(s * RS + k * 25, 25), :])

    plsc.subcore_barrier()

    srcv0 = src_slab[pl.ds(0, BK)]
    dstv0 = dst_slab[pl.ds(0, BK)]
    pltpu.async_copy(xl_hbm.at[srcv0], xl_rows2.at[0], sem_xl.at[0])
    pltpu.async_copy(xr_hbm.at[dstv0], xr_rows2.at[0], sem_xr.at[0])

    @pl.loop(0, NB)
    def _(g):
        p = g & 1
        pn = 1 - p
        base = wid * EW + g * BK
        srcv = src_slab[pl.ds(g * BK, BK)]
        dstv = dst_slab[pl.ds(g * BK, BK)]
        pltpu.make_async_copy(xl_hbm.at[srcv], xl_rows2.at[p], sem_xl.at[p]).wait()
        pltpu.make_async_copy(xr_hbm.at[dstv], xr_rows2.at[p], sem_xr.at[p]).wait()

        @pl.when(g < NB - 1)
        def _():
            srcv2 = src_slab[pl.ds((g + 1) * BK, BK)]
            dstv2 = dst_slab[pl.ds((g + 1) * BK, BK)]
            pltpu.async_copy(xl_hbm.at[srcv2], xl_rows2.at[pn], sem_xl.at[pn])
            pltpu.async_copy(xr_hbm.at[dstv2], xr_rows2.at[pn], sem_xr.at[pn])

        @pl.when(g >= 2)
        def _():
            pltpu.make_async_copy(ex_stage2.at[p], exq_hbm.at[pl.ds(base, BK)],
                                  sem_ex.at[p]).wait()

        splat_p = _splat_i(0) + p
        for h in range(8):
            def fbody(jo, acc, h=h):
                attv = att_buf[h, pl.ds(jo * 16, 16)]
                for jj in range(16):
                    col = _splat_i(h * 64 + jo * 16 + jj)
                    xlv = plsc.load_gather(xl_rows2, [splat_p, lanes, col])
                    xrv = plsc.load_gather(xr_rows2, [splat_p, lanes, col])
                    m = xlv + xrv
                    lk = jnp.maximum(m, SLOPE * m)
                    acc = acc + attv[jj] * lk
                return acc
            acc = lax.fori_loop(0, 4, fbody, _zero16())
            exh = jnp.exp(acc)
            plsc.store_scatter(ex_stage2, [splat_p, lanes, _splat_i(h)], exh)

        pltpu.sync_copy(ex_stage2.at[p], accum.at[dstv], add=True)
        pltpu.async_copy(ex_stage2.at[p], exq_hbm.at[pl.ds(base, BK)],
                         sem_ex.at[p])

    for p in range(2):
        pltpu.make_async_copy(ex_stage2.at[p], exq_hbm.at[pl.ds(wid * EW, BK)],
                              sem_ex.at[p]).wait()

    plsc.subcore_barrier()
    pltpu.sync_copy(accum.at[pl.ds(s * RS, RS), :],
                    den_hbm.at[c, pl.ds(s * RS, RS)])


def _sc1a(xl, xr, src, dst, att, N, E):
    EW = E // NW
    mesh = plsc.VectorSubcoreMesh(core_axis_name="c", subcore_axis_name="s")
    return pl.kernel(
        functools.partial(_sc1a_body, N, E),
        out_type=(
            jax.ShapeDtypeStruct((NC, N, 16), _F32),
            jax.ShapeDtypeStruct((E, 16), _F32),
        ),
        mesh=mesh,
        compiler_params=_SC_PARAMS,
        scratch_types=[
            pltpu.VMEM_SHARED((N, 16), _F32),
            pltpu.VMEM((EW,), _I32),
            pltpu.VMEM((EW,), _I32),
            pltpu.VMEM((2, BK, 512), _F32),
            pltpu.VMEM((2, BK, 512), _F32),
            pltpu.VMEM((2, BK, 16), _F32),
            pltpu.VMEM((8, 64), _F32),
            pltpu.VMEM((25, 16), _F32),
            pltpu.SemaphoreType.DMA((2,)),
            pltpu.SemaphoreType.DMA((2,)),
            pltpu.SemaphoreType.DMA((2,)),
        ],
    )(xl, xr, src, dst, att)


def _sc1b_body(N, E, xlc_hbm, src_hbm, dst_hbm, exq_hbm,
               un_hbm,
               accum, src_slab, dst_slab, xc_rows2, ex_stage2, row_buf,
               zero_buf, sem_xc, sem_ex):
    c = lax.axis_index("c")
    s = lax.axis_index("s")
    wid = c * NS + s
    EW = E // NW
    NB = EW // BK
    RS = N // NS
    lanes = _lanes()

    pltpu.sync_copy(src_hbm.at[pl.ds(wid * EW, EW)], src_slab)
    pltpu.sync_copy(dst_hbm.at[pl.ds(wid * EW, EW)], dst_slab)

    @pl.loop(0, 25)
    def _(i):
        for j in range(8):
            zero_buf[i, pl.ds(j * 16, 16)] = _zero16()

    def zero_accum():
        @pl.loop(0, RS // 25)
        def _(k):
            pltpu.sync_copy(zero_buf, accum.at[pl.ds(s * RS + k * 25, 25), :])

    zero_accum()
    plsc.subcore_barrier()

    for ch in range(4):
        srcv0 = src_slab[pl.ds(0, BK)]
        pltpu.async_copy(xlc_hbm.at[ch].at[srcv0], xc_rows2.at[0], sem_xc.at[0])
        pltpu.async_copy(exq_hbm.at[pl.ds(wid * EW, BK)], ex_stage2.at[0],
                         sem_ex.at[0])

        @pl.loop(0, NB)
        def _(g, ch=ch):
            p = g & 1
            pn = 1 - p
            base = wid * EW + g * BK
            srcv = src_slab[pl.ds(g * BK, BK)]
            dstv = dst_slab[pl.ds(g * BK, BK)]
            pltpu.make_async_copy(xlc_hbm.at[ch].at[srcv], xc_rows2.at[p],
                                  sem_xc.at[p]).wait()
            pltpu.make_async_copy(exq_hbm.at[pl.ds(base, BK)], ex_stage2.at[p],
                                  sem_ex.at[p]).wait()

            @pl.when(g < NB - 1)
            def _():
                srcv2 = src_slab[pl.ds((g + 1) * BK, BK)]
                pltpu.async_copy(xlc_hbm.at[ch].at[srcv2], xc_rows2.at[pn],
                                 sem_xc.at[pn])
                pltpu.async_copy(exq_hbm.at[pl.ds(base + BK, BK)],
                                 ex_stage2.at[pn], sem_ex.at[pn])

            @pl.loop(0, BK)
            def _(e, ch=ch):
                exv = ex_stage2[p, e, pl.ds(0, 16)]
                for j in range(8):
                    row_buf[e, pl.ds(j * 16, 16)] = (
                        xc_rows2[p, e, pl.ds(j * 16, 16)] * exv[2 * ch + j // 4])

            pltpu.sync_copy(row_buf, accum.at[dstv], add=True)

        plsc.subcore_barrier()
        pltpu.sync_copy(accum.at[pl.ds(s * RS, RS), :],
                        un_hbm.at[c, ch, pl.ds(s * RS, RS)])
        if ch < 3:
            zero_accum()
        plsc.subcore_barrier()


def _sc1b(xlc, src, dst, exq, N, E):
    EW = E // NW
    mesh = plsc.VectorSubcoreMesh(core_axis_name="c", subcore_axis_name="s")
    return pl.kernel(
        functools.partial(_sc1b_body, N, E),
        out_type=jax.ShapeDtypeStruct((NC, 4, N, 128), _F32),
        mesh=mesh,
        compiler_params=_SC_PARAMS,
        scratch_types=[
            pltpu.VMEM_SHARED((N, 128), _F32),
            pltpu.VMEM((EW,), _I32),
            pltpu.VMEM((EW,), _I32),
            pltpu.VMEM((2, BK, 128), _F32),
            pltpu.VMEM((2, BK, 16), _F32),
            pltpu.VMEM((BK, 128), _F32),
            pltpu.VMEM((25, 128), _F32),
            pltpu.SemaphoreType.DMA((2,)),
            pltpu.SemaphoreType.DMA((2,)),
        ],
    )(xlc, src, dst, exq)


def _tc2_body(u_ref, d_ref, b1_ref, wl2_ref, bl2_ref, wr2_ref, br2_ref,
              xl2_ref, xr2_ref):
    u = u_ref[...]
    us = u[0] + u[1]
    d = d_ref[...]
    dsum = d[0] + d[1]
    cols = []
    for ch in range(4):
        for k in range(2):
            dh = dsum[:, 2 * ch + k:2 * ch + k + 1] + EPS
            cols.append(us[ch][:, 64 * k:64 * (k + 1)] / dh)
    h = jnp.concatenate(cols, axis=1) + b1_ref[...]
    h = jnp.maximum(h, 0.0)
    xl2_ref[...] = jnp.dot(h, wl2_ref[...], preferred_element_type=_F32,
                           precision=lax.Precision.HIGHEST) + bl2_ref[...]
    xr2_ref[...] = jnp.dot(h, wr2_ref[...], preferred_element_type=_F32,
                           precision=lax.Precision.HIGHEST) + br2_ref[...]


def _tc2(un, den, b1, wl2, bl2, wr2, br2, N, HF):
    BN = 1000
    return pl.pallas_call(
        _tc2_body,
        grid=(N // BN,),
        in_specs=[
            pl.BlockSpec((NC, 4, BN, 128), lambda i: (0, 0, i, 0)),
            pl.BlockSpec((NC, BN, 16), lambda i: (0, i, 0)),
            pl.BlockSpec((1, HF), lambda i: (0, 0)),
            pl.BlockSpec((HF, 1), lambda i: (0, 0)),
            pl.BlockSpec((1, 1), lambda i: (0, 0)),
            pl.BlockSpec((HF, 1), lambda i: (0, 0)),
            pl.BlockSpec((1, 1), lambda i: (0, 0)),
        ],
        out_specs=[
            pl.BlockSpec((BN, 1), lambda i: (i, 0)),
            pl.BlockSpec((BN, 1), lambda i: (i, 0)),
        ],
        out_shape=[
            jax.ShapeDtypeStruct((N, 1), _F32),
            jax.ShapeDtypeStruct((N, 1), _F32),
        ],
    )(un, den, b1, wl2, bl2, wr2, br2)


def _sc2a_body(N, E, xl2_hbm, xr2_hbm, src_hbm, dst_hbm, att2_hbm,
               parts_hbm, ex2q_hbm,
               accum2, xl2_buf, xr2_buf, att2_buf, src_slab, dst_slab,
               ex2_slab, row2_buf, zero2_buf):
    c = lax.axis_index("c")
    s = lax.axis_index("s")
    wid = c * NS + s
    EW = E // NW
    NB = EW // BK
    RS = N // NS
    ZR = 125
    lanes = _lanes()

    pltpu.sync_copy(xl2_hbm, xl2_buf)
    pltpu.sync_copy(xr2_hbm, xr2_buf)
    pltpu.sync_copy(att2_hbm, att2_buf)
    pltpu.sync_copy(src_hbm.at[pl.ds(wid * EW, EW)], src_slab)
    pltpu.sync_copy(dst_hbm.at[pl.ds(wid * EW, EW)], dst_slab)

    @pl.loop(0, ZR)
    def _(i):
        zero2_buf[i, pl.ds(0, 16)] = _zero16()

    @pl.loop(0, BK)
    def _(e):
        row2_buf[e, pl.ds(0, 16)] = _zero16()

    @pl.loop(0, RS // ZR)
    def _(k):
        pltpu.sync_copy(zero2_buf, accum2.at[pl.ds(s * RS + k * ZR, ZR), :])

    plsc.subcore_barrier()
    attv = att2_buf[...]

    @pl.loop(0, NB)
    def _(b):
        srcv = src_slab[pl.ds(b * BK, BK)]
        dstv = dst_slab[pl.ds(b * BK, BK)]
        xls = plsc.load_gather(xl2_buf, [srcv])
        xrd = plsc.load_gather(xr2_buf, [dstv])
        m = xls + xrd
        lk = jnp.maximum(m, SLOPE * m)
        ex2 = jnp.exp(attv * lk)
        plsc.store_scatter(row2_buf, [lanes, _splat_i(0)], ex2 * xls)
        plsc.store_scatter(row2_buf, [lanes, _splat_i(1)], ex2)
        ex2_slab[pl.ds(b * BK, BK)] = ex2
        pltpu.sync_copy(row2_buf, accum2.at[dstv], add=True)

    pltpu.sync_copy(ex2_slab, ex2q_hbm.at[pl.ds(wid * EW, EW)])
    plsc.subcore_barrier()
    pltpu.sync_copy(accum2.at[pl.ds(s * RS, RS), :],
                    parts_hbm.at[c, pl.ds(s * RS, RS)])


def _sc2a(xl2, xr2, src, dst, att2v, N, E):
    EW = E // NW
    mesh = plsc.VectorSubcoreMesh(core_axis_name="c", subcore_axis_name="s")
    return pl.kernel(
        functools.partial(_sc2a_body, N, E),
        out_type=(
            jax.ShapeDtypeStruct((NC, N, 16), _F32),
            jax.ShapeDtypeStruct((E,), _F32),
        ),
        mesh=mesh,
        compiler_params=_SC_PARAMS,
        scratch_types=[
            pltpu.VMEM_SHARED((N, 16), _F32),
            pltpu.VMEM((N,), _F32),
            pltpu.VMEM((N,), _F32),
            pltpu.VMEM((16,), _F32),
            pltpu.VMEM((EW,), _I32),
            pltpu.VMEM((EW,), _I32),
            pltpu.VMEM((EW,), _F32),
            pltpu.VMEM((BK, 16), _F32),
            pltpu.VMEM((125, 16), _F32),
        ],
    )(xl2, xr2, src, dst, att2v)


def _tc2b_body(p_ref, b2_ref, out2_ref, dtot_ref):
    p = p_ref[...]
    u = p[0, :, 0:1] + p[1, :, 0:1]
    dd = p[0, :, 1:2] + p[1, :, 1:2] + EPS
    out2_ref[...] = u / dd + b2_ref[...]
    dtot_ref[...] = dd


def _tc2b(parts, b2, N):
    BN = 1000
    return pl.pallas_call(
        _tc2b_body,
        grid=(N // BN,),
        in_specs=[
            pl.BlockSpec((NC, BN, 16), lambda i: (0, i, 0)),
            pl.BlockSpec((1, 1), lambda i: (0, 0)),
        ],
        out_specs=[
            pl.BlockSpec((BN, 1), lambda i: (i, 0)),
            pl.BlockSpec((BN, 1), lambda i: (i, 0)),
        ],
        out_shape=[
            jax.ShapeDtypeStruct((N, 1), _F32),
            jax.ShapeDtypeStruct((N, 1), _F32),
        ],
    )(parts, b2)


def _sc2b_body(N, E, dtot_hbm, dst_hbm, ex2q_hbm, a2_hbm,
               dt_buf, dst_slab, e2_slab, a2_slab):
    c = lax.axis_index("c")
    s = lax.axis_index("s")
    wid = c * NS + s
    EW = E // NW
    NB = EW // BK

    pltpu.sync_copy(dtot_hbm, dt_buf)
    pltpu.sync_copy(dst_hbm.at[pl.ds(wid * EW, EW)], dst_slab)
    pltpu.sync_copy(ex2q_hbm.at[pl.ds(wid * EW, EW)], e2_slab)

    @pl.loop(0, NB)
    def _(b):
        dstv = dst_slab[pl.ds(b * BK, BK)]
        dv = plsc.load_gather(dt_buf, [dstv])
        a2_slab[pl.ds(b * BK, BK)] = e2_slab[pl.ds(b * BK, BK)] / dv

    pltpu.sync_copy(a2_slab, a2_hbm.at[pl.ds(wid * EW, EW)])


def _sc2b(dtot, dst, ex2q, N, E):
    EW = E // NW
    mesh = plsc.VectorSubcoreMesh(core_axis_name="c", subcore_axis_name="s")
    return pl.kernel(
        functools.partial(_sc2b_body, N, E),
        out_type=jax.ShapeDtypeStruct((E,), _F32),
        mesh=mesh,
        compiler_params=_SC_PARAMS,
        scratch_types=[
            pltpu.VMEM((N,), _F32),
            pltpu.VMEM((EW,), _I32),
            pltpu.VMEM((EW,), _F32),
            pltpu.VMEM((EW,), _F32),
        ],
    )(dtot, dst, ex2q)


def kernel(x, edge_index, Wl1, bl1, Wr1, br1, att1, bias1,
           Wl2, bl2, Wr2, br2, att2, bias2):
    N, D = x.shape
    E = edge_index.shape[1]
    HF = Wl1.shape[1]
    src = edge_index[0]
    dst = edge_index[1]

    xl, xr, xlc = _tc1(x, Wl1, bl1.reshape(1, HF), Wr1, br1.reshape(1, HF),
                       N, D, HF)
    den, exq = _sc1a(xl, xr, src, dst, att1, N, E)
    un = _sc1b(xlc, src, dst, exq, N, E)
    xl2, xr2 = _tc2(un, den, bias1.reshape(1, HF), Wl2, bl2.reshape(1, 1),
                    Wr2, br2.reshape(1, 1), N, HF)
    att2v = jnp.full((16,), att2[0, 0], _F32)
    parts, ex2q = _sc2a(xl2.reshape(N), xr2.reshape(N), src, dst, att2v, N, E)
    out2, dtot = _tc2b(parts, bias2.reshape(1, 1), N)
    a2 = _sc2b(dtot.reshape(N), dst, ex2q, N, E)
    return out2, edge_index, a2.reshape(E, 1)


# speedup vs baseline: 6.9121x; 1.0361x over previous
import functools

import jax
import jax.numpy as jnp
from jax import lax
from jax.experimental import pallas as pl
from jax.experimental.pallas import tpu as pltpu
from jax.experimental.pallas import tpu_sc as plsc

NS = 16
NC = 2
NW = NC * NS
BK = 16
SLOPE = 0.2
EPS = 1e-16

_F32 = jnp.float32
_I32 = jnp.int32

_SC_PARAMS = pltpu.CompilerParams(use_tc_tiling_on_sc=False,
                                  needs_layout_passes=False)


def _lanes():
    return lax.iota(_I32, 16)


def _splat_i(v):
    return jnp.zeros((16,), _I32) + v


def _zero16():
    return jnp.zeros((16,), _F32)


def _tc1_body(x_ref, wl_ref, bl_ref, wr_ref, br_ref, xl_ref, xr_ref, xlc_ref):
    x = x_ref[...]
    xl = jnp.dot(x, wl_ref[...], preferred_element_type=_F32,
                 precision=lax.Precision.HIGHEST) + bl_ref[...]
    xr = jnp.dot(x, wr_ref[...], preferred_element_type=_F32,
                 precision=lax.Precision.HIGHEST) + br_ref[...]
    xl_ref[...] = xl
    xr_ref[...] = xr
    for c in range(4):
        xlc_ref[c] = xl[:, c * 128:(c + 1) * 128]


def _tc1(x, wl, bl, wr, br, N, D, HF):
    BN = 1000
    return pl.pallas_call(
        _tc1_body,
        grid=(N // BN,),
        in_specs=[
            pl.BlockSpec((BN, D), lambda i: (i, 0)),
            pl.BlockSpec((D, HF), lambda i: (0, 0)),
            pl.BlockSpec((1, HF), lambda i: (0, 0)),
            pl.BlockSpec((D, HF), lambda i: (0, 0)),
            pl.BlockSpec((1, HF), lambda i: (0, 0)),
        ],
        out_specs=[
            pl.BlockSpec((BN, HF), lambda i: (i, 0)),
            pl.BlockSpec((BN, HF), lambda i: (i, 0)),
            pl.BlockSpec((4, BN, 128), lambda i: (0, i, 0)),
        ],
        out_shape=[
            jax.ShapeDtypeStruct((N, HF), _F32),
            jax.ShapeDtypeStruct((N, HF), _F32),
            jax.ShapeDtypeStruct((4, N, 128), _F32),
        ],
    )(x, wl, bl, wr, br)


def _sc1a_body(N, E, xl_hbm, xr_hbm, src_hbm, dst_hbm, att_hbm,
               den_hbm, exq_hbm,
               accum, src_slab, dst_slab, xl_rows2, xr_rows2,
               ex_stage2, att_buf, zero_buf, sem_xl, sem_xr, sem_ex, sem_sca):
    c = lax.axis_index("c")
    s = lax.axis_index("s")
    wid = c * NS + s
    EW = E // NW
    NB = EW // BK
    RS = N // NS
    lanes = _lanes()

    pltpu.sync_copy(att_hbm, att_buf)
    pltpu.sync_copy(src_hbm.at[pl.ds(wid * EW, EW)], src_slab)
    pltpu.sync_copy(dst_hbm.at[pl.ds(wid * EW, EW)], dst_slab)

    @pl.loop(0, 25)
    def _(i):
        zero_buf[i, pl.ds(0, 16)] = _zero16()

    @pl.loop(0, BK)
    def _(e):
        for p in range(2):
            ex_stage2[p, e, pl.ds(0, 16)] = _zero16()

    @pl.loop(0, RS // 25)
    def _(k):
        pltpu.sync_copy(zero_buf, accum.at[pl.ds(s * RS + k * 25, 25), :])

    plsc.subcore_barrier()

    for q in range(3):
        srcv0 = src_slab[pl.ds(q * BK, BK)]
        dstv0 = dst_slab[pl.ds(q * BK, BK)]
        pltpu.async_copy(xl_hbm.at[srcv0], xl_rows2.at[q], sem_xl.at[q])
        pltpu.async_copy(xr_hbm.at[dstv0], xr_rows2.at[q], sem_xr.at[q])

    @pl.loop(0, NB)
    def _(g):
        p = g & 3
        base = wid * EW + g * BK
        srcv = src_slab[pl.ds(g * BK, BK)]
        dstv = dst_slab[pl.ds(g * BK, BK)]
        pltpu.make_async_copy(xl_hbm.at[srcv], xl_rows2.at[p], sem_xl.at[p]).wait()
        pltpu.make_async_copy(xr_hbm.at[dstv], xr_rows2.at[p], sem_xr.at[p]).wait()

        @pl.when(g < NB - 3)
        def _():
            pf = (g + 3) & 3
            srcv2 = src_slab[pl.ds((g + 3) * BK, BK)]
            dstv2 = dst_slab[pl.ds((g + 3) * BK, BK)]
            pltpu.async_copy(xl_hbm.at[srcv2], xl_rows2.at[pf], sem_xl.at[pf])
            pltpu.async_copy(xr_hbm.at[dstv2], xr_rows2.at[pf], sem_xr.at[pf])

        pe = g & 1

        @pl.when(g >= 2)
        def _():
            pltpu.make_async_copy(ex_stage2.at[pe], exq_hbm.at[pl.ds(base, BK)],
                                  sem_ex.at[pe]).wait()
            pltpu.make_async_copy(ex_stage2.at[pe], accum.at[dstv],
                                  sem_sca.at[pe]).wait()

        splat_p = _splat_i(0) + p
        for h in range(8):
            def fbody(jo, acc, h=h):
                attv = att_buf[h, pl.ds(jo * 16, 16)]
                for jj in range(16):
                    col = _splat_i(h * 64 + jo * 16 + jj)
                    xlv = plsc.load_gather(xl_rows2, [splat_p, lanes, col])
                    xrv = plsc.load_gather(xr_rows2, [splat_p, lanes, col])
                    m = xlv + xrv
                    lk = jnp.maximum(m, SLOPE * m)
                    acc = acc + attv[jj] * lk
                return acc
            acc = lax.fori_loop(0, 4, fbody, _zero16())
            exh = jnp.exp(acc)
            plsc.store_scatter(ex_stage2, [_splat_i(0) + pe, lanes,
                                           _splat_i(h)], exh)

        pltpu.async_copy(ex_stage2.at[pe], accum.at[dstv], sem_sca.at[pe],
                         add=True)
        pltpu.async_copy(ex_stage2.at[pe], exq_hbm.at[pl.ds(base, BK)],
                         sem_ex.at[pe])

    dstv_last = dst_slab[pl.ds((NB - 1) * BK, BK)]
    for p in range(2):
        pltpu.make_async_copy(ex_stage2.at[p], exq_hbm.at[pl.ds(wid * EW, BK)],
                              sem_ex.at[p]).wait()
        pltpu.make_async_copy(ex_stage2.at[p], accum.at[dstv_last],
                              sem_sca.at[p]).wait()

    plsc.subcore_barrier()
    pltpu.sync_copy(accum.at[pl.ds(s * RS, RS), :],
                    den_hbm.at[c, pl.ds(s * RS, RS)])


def _sc1a(xl, xr, src, dst, att, N, E):
    EW = E // NW
    mesh = plsc.VectorSubcoreMesh(core_axis_name="c", subcore_axis_name="s")
    return pl.kernel(
        functools.partial(_sc1a_body, N, E),
        out_type=(
            jax.ShapeDtypeStruct((NC, N, 16), _F32),
            jax.ShapeDtypeStruct((E, 16), _F32),
        ),
        mesh=mesh,
        compiler_params=_SC_PARAMS,
        scratch_types=[
            pltpu.VMEM_SHARED((N, 16), _F32),
            pltpu.VMEM((EW,), _I32),
            pltpu.VMEM((EW,), _I32),
            pltpu.VMEM((4, BK, 512), _F32),
            pltpu.VMEM((4, BK, 512), _F32),
            pltpu.VMEM((2, BK, 16), _F32),
            pltpu.VMEM((8, 64), _F32),
            pltpu.VMEM((25, 16), _F32),
            pltpu.SemaphoreType.DMA((4,)),
            pltpu.SemaphoreType.DMA((4,)),
            pltpu.SemaphoreType.DMA((2,)),
            pltpu.SemaphoreType.DMA((2,)),
        ],
    )(xl, xr, src, dst, att)


def _sc1b_body(N, E, xlc_hbm, src_hbm, dst_hbm, exq_hbm,
               un_hbm,
               accum, src_slab, dst_slab, xc_rows2, ex_stage2, row_buf2,
               zero_buf, sem_xc, sem_ex, sem_sc):
    c = lax.axis_index("c")
    s = lax.axis_index("s")
    wid = c * NS + s
    EW = E // NW
    NB = EW // BK
    RS = N // NS
    lanes = _lanes()

    pltpu.sync_copy(src_hbm.at[pl.ds(wid * EW, EW)], src_slab)
    pltpu.sync_copy(dst_hbm.at[pl.ds(wid * EW, EW)], dst_slab)

    @pl.loop(0, 25)
    def _(i):
        for j in range(8):
            zero_buf[i, pl.ds(j * 16, 16)] = _zero16()

    def zero_accum():
        @pl.loop(0, RS // 25)
        def _(k):
            pltpu.sync_copy(zero_buf, accum.at[pl.ds(s * RS + k * 25, 25), :])

    zero_accum()
    plsc.subcore_barrier()

    for ch in range(4):
        for q in range(3):
            srcv0 = src_slab[pl.ds(q * BK, BK)]
            pltpu.async_copy(xlc_hbm.at[ch].at[srcv0], xc_rows2.at[q],
                             sem_xc.at[q])
            pltpu.async_copy(exq_hbm.at[pl.ds(wid * EW + q * BK, BK)],
                             ex_stage2.at[q], sem_ex.at[q])

        @pl.loop(0, NB)
        def _(g, ch=ch):
            p = g & 3
            pe = g & 1
            base = wid * EW + g * BK
            srcv = src_slab[pl.ds(g * BK, BK)]
            dstv = dst_slab[pl.ds(g * BK, BK)]
            pltpu.make_async_copy(xlc_hbm.at[ch].at[srcv], xc_rows2.at[p],
                                  sem_xc.at[p]).wait()
            pltpu.make_async_copy(exq_hbm.at[pl.ds(base, BK)], ex_stage2.at[p],
                                  sem_ex.at[p]).wait()

            @pl.when(g >= 2)
            def _():
                pltpu.make_async_copy(row_buf2.at[pe], accum.at[dstv],
                                      sem_sc.at[pe]).wait()

            @pl.when(g < NB - 3)
            def _():
                pf = (g + 3) & 3
                srcv2 = src_slab[pl.ds((g + 3) * BK, BK)]
                pltpu.async_copy(xlc_hbm.at[ch].at[srcv2], xc_rows2.at[pf],
                                 sem_xc.at[pf])
                pltpu.async_copy(exq_hbm.at[pl.ds(base + 3 * BK, BK)],
                                 ex_stage2.at[pf], sem_ex.at[pf])

            @pl.loop(0, BK)
            def _(e, ch=ch):
                exv = ex_stage2[p, e, pl.ds(0, 16)]
                for j in range(8):
                    row_buf2[pe, e, pl.ds(j * 16, 16)] = (
                        xc_rows2[p, e, pl.ds(j * 16, 16)] * exv[2 * ch + j // 4])

            pltpu.async_copy(row_buf2.at[pe], accum.at[dstv], sem_sc.at[pe],
                             add=True)

        dstv_last = dst_slab[pl.ds((NB - 1) * BK, BK)]
        for p in range(2):
            pltpu.make_async_copy(row_buf2.at[p], accum.at[dstv_last],
                                  sem_sc.at[p]).wait()
        plsc.subcore_barrier()
        pltpu.sync_copy(accum.at[pl.ds(s * RS, RS), :],
                        un_hbm.at[c, ch, pl.ds(s * RS, RS)])
        if ch < 3:
            zero_accum()
        plsc.subcore_barrier()


def _sc1b(xlc, src, dst, exq, N, E):
    EW = E // NW
    mesh = plsc.VectorSubcoreMesh(core_axis_name="c", subcore_axis_name="s")
    return pl.kernel(
        functools.partial(_sc1b_body, N, E),
        out_type=jax.ShapeDtypeStruct((NC, 4, N, 128), _F32),
        mesh=mesh,
        compiler_params=_SC_PARAMS,
        scratch_types=[
            pltpu.VMEM_SHARED((N, 128), _F32),
            pltpu.VMEM((EW,), _I32),
            pltpu.VMEM((EW,), _I32),
            pltpu.VMEM((4, BK, 128), _F32),
            pltpu.VMEM((4, BK, 16), _F32),
            pltpu.VMEM((2, BK, 128), _F32),
            pltpu.VMEM((25, 128), _F32),
            pltpu.SemaphoreType.DMA((4,)),
            pltpu.SemaphoreType.DMA((4,)),
            pltpu.SemaphoreType.DMA((2,)),
        ],
    )(xlc, src, dst, exq)


def _tc2_body(u_ref, d_ref, b1_ref, wl2_ref, bl2_ref, wr2_ref, br2_ref,
              xl2_ref, xr2_ref):
    u = u_ref[...]
    us = u[0] + u[1]
    d = d_ref[...]
    dsum = d[0] + d[1]
    cols = []
    for ch in range(4):
        for k in range(2):
            dh = dsum[:, 2 * ch + k:2 * ch + k + 1] + EPS
            cols.append(us[ch][:, 64 * k:64 * (k + 1)] / dh)
    h = jnp.concatenate(cols, axis=1) + b1_ref[...]
    h = jnp.maximum(h, 0.0)
    xl2_ref[...] = jnp.dot(h, wl2_ref[...], preferred_element_type=_F32,
                           precision=lax.Precision.HIGHEST) + bl2_ref[...]
    xr2_ref[...] = jnp.dot(h, wr2_ref[...], preferred_element_type=_F32,
                           precision=lax.Precision.HIGHEST) + br2_ref[...]


def _tc2(un, den, b1, wl2, bl2, wr2, br2, N, HF):
    BN = 1000
    return pl.pallas_call(
        _tc2_body,
        grid=(N // BN,),
        in_specs=[
            pl.BlockSpec((NC, 4, BN, 128), lambda i: (0, 0, i, 0)),
            pl.BlockSpec((NC, BN, 16), lambda i: (0, i, 0)),
            pl.BlockSpec((1, HF), lambda i: (0, 0)),
            pl.BlockSpec((HF, 1), lambda i: (0, 0)),
            pl.BlockSpec((1, 1), lambda i: (0, 0)),
            pl.BlockSpec((HF, 1), lambda i: (0, 0)),
            pl.BlockSpec((1, 1), lambda i: (0, 0)),
        ],
        out_specs=[
            pl.BlockSpec((BN, 1), lambda i: (i, 0)),
            pl.BlockSpec((BN, 1), lambda i: (i, 0)),
        ],
        out_shape=[
            jax.ShapeDtypeStruct((N, 1), _F32),
            jax.ShapeDtypeStruct((N, 1), _F32),
        ],
    )(un, den, b1, wl2, bl2, wr2, br2)


def _sc2a_body(N, E, xl2_hbm, xr2_hbm, src_hbm, dst_hbm, att2_hbm,
               parts_hbm, ex2q_hbm,
               accum2, xl2_buf, xr2_buf, att2_buf, src_slab, dst_slab,
               ex2_slab, row2_buf, zero2_buf):
    c = lax.axis_index("c")
    s = lax.axis_index("s")
    wid = c * NS + s
    EW = E // NW
    NB = EW // BK
    RS = N // NS
    ZR = 125
    lanes = _lanes()

    pltpu.sync_copy(xl2_hbm, xl2_buf)
    pltpu.sync_copy(xr2_hbm, xr2_buf)
    pltpu.sync_copy(att2_hbm, att2_buf)
    pltpu.sync_copy(src_hbm.at[pl.ds(wid * EW, EW)], src_slab)
    pltpu.sync_copy(dst_hbm.at[pl.ds(wid * EW, EW)], dst_slab)

    @pl.loop(0, ZR)
    def _(i):
        zero2_buf[i, pl.ds(0, 16)] = _zero16()

    @pl.loop(0, BK)
    def _(e):
        row2_buf[e, pl.ds(0, 16)] = _zero16()

    @pl.loop(0, RS // ZR)
    def _(k):
        pltpu.sync_copy(zero2_buf, accum2.at[pl.ds(s * RS + k * ZR, ZR), :])

    plsc.subcore_barrier()
    attv = att2_buf[...]

    @pl.loop(0, NB)
    def _(b):
        srcv = src_slab[pl.ds(b * BK, BK)]
        dstv = dst_slab[pl.ds(b * BK, BK)]
        xls = plsc.load_gather(xl2_buf, [srcv])
        xrd = plsc.load_gather(xr2_buf, [dstv])
        m = xls + xrd
        lk = jnp.maximum(m, SLOPE * m)
        ex2 = jnp.exp(attv * lk)
        plsc.store_scatter(row2_buf, [lanes, _splat_i(0)], ex2 * xls)
        plsc.store_scatter(row2_buf, [lanes, _splat_i(1)], ex2)
        ex2_slab[pl.ds(b * BK, BK)] = ex2
        pltpu.sync_copy(row2_buf, accum2.at[dstv], add=True)

    pltpu.sync_copy(ex2_slab, ex2q_hbm.at[pl.ds(wid * EW, EW)])
    plsc.subcore_barrier()
    pltpu.sync_copy(accum2.at[pl.ds(s * RS, RS), :],
                    parts_hbm.at[c, pl.ds(s * RS, RS)])


def _sc2a(xl2, xr2, src, dst, att2v, N, E):
    EW = E // NW
    mesh = plsc.VectorSubcoreMesh(core_axis_name="c", subcore_axis_name="s")
    return pl.kernel(
        functools.partial(_sc2a_body, N, E),
        out_type=(
            jax.ShapeDtypeStruct((NC, N, 16), _F32),
            jax.ShapeDtypeStruct((E,), _F32),
        ),
        mesh=mesh,
        compiler_params=_SC_PARAMS,
        scratch_types=[
            pltpu.VMEM_SHARED((N, 16), _F32),
            pltpu.VMEM((N,), _F32),
            pltpu.VMEM((N,), _F32),
            pltpu.VMEM((16,), _F32),
            pltpu.VMEM((EW,), _I32),
            pltpu.VMEM((EW,), _I32),
            pltpu.VMEM((EW,), _F32),
            pltpu.VMEM((BK, 16), _F32),
            pltpu.VMEM((125, 16), _F32),
        ],
    )(xl2, xr2, src, dst, att2v)


def _tc2b_body(p_ref, b2_ref, out2_ref, dtot_ref):
    p = p_ref[...]
    u = p[0, :, 0:1] + p[1, :, 0:1]
    dd = p[0, :, 1:2] + p[1, :, 1:2] + EPS
    out2_ref[...] = u / dd + b2_ref[...]
    dtot_ref[...] = dd


def _tc2b(parts, b2, N):
    BN = 1000
    return pl.pallas_call(
        _tc2b_body,
        grid=(N // BN,),
        in_specs=[
            pl.BlockSpec((NC, BN, 16), lambda i: (0, i, 0)),
            pl.BlockSpec((1, 1), lambda i: (0, 0)),
        ],
        out_specs=[
            pl.BlockSpec((BN, 1), lambda i: (i, 0)),
            pl.BlockSpec((BN, 1), lambda i: (i, 0)),
        ],
        out_shape=[
            jax.ShapeDtypeStruct((N, 1), _F32),
            jax.ShapeDtypeStruct((N, 1), _F32),
        ],
    )(parts, b2)


def _sc2b_body(N, E, dtot_hbm, dst_hbm, ex2q_hbm, a2_hbm,
               dt_buf, dst_slab, e2_slab, a2_slab):
    c = lax.axis_index("c")
    s = lax.axis_index("s")
    wid = c * NS + s
    EW = E // NW
    NB = EW // BK

    pltpu.sync_copy(dtot_hbm, dt_buf)
    pltpu.sync_copy(dst_hbm.at[pl.ds(wid * EW, EW)], dst_slab)
    pltpu.sync_copy(ex2q_hbm.at[pl.ds(wid * EW, EW)], e2_slab)

    @pl.loop(0, NB)
    def _(b):
        dstv = dst_slab[pl.ds(b * BK, BK)]
        dv = plsc.load_gather(dt_buf, [dstv])
        a2_slab[pl.ds(b * BK, BK)] = e2_slab[pl.ds(b * BK, BK)] / dv

    pltpu.sync_copy(a2_slab, a2_hbm.at[pl.ds(wid * EW, EW)])


def _sc2b(dtot, dst, ex2q, N, E):
    EW = E // NW
    mesh = plsc.VectorSubcoreMesh(core_axis_name="c", subcore_axis_name="s")
    return pl.kernel(
        functools.partial(_sc2b_body, N, E),
        out_type=jax.ShapeDtypeStruct((E,), _F32),
        mesh=mesh,
        compiler_params=_SC_PARAMS,
        scratch_types=[
            pltpu.VMEM((N,), _F32),
            pltpu.VMEM((EW,), _I32),
            pltpu.VMEM((EW,), _F32),
            pltpu.VMEM((EW,), _F32),
        ],
    )(dtot, dst, ex2q)


def kernel(x, edge_index, Wl1, bl1, Wr1, br1, att1, bias1,
           Wl2, bl2, Wr2, br2, att2, bias2):
    N, D = x.shape
    E = edge_index.shape[1]
    HF = Wl1.shape[1]
    src = edge_index[0]
    dst = edge_index[1]

    xl, xr, xlc = _tc1(x, Wl1, bl1.reshape(1, HF), Wr1, br1.reshape(1, HF),
                       N, D, HF)
    den, exq = _sc1a(xl, xr, src, dst, att1, N, E)
    un = _sc1b(xlc, src, dst, exq, N, E)
    xl2, xr2 = _tc2(un, den, bias1.reshape(1, HF), Wl2, bl2.reshape(1, 1),
                    Wr2, br2.reshape(1, 1), N, HF)
    att2v = jnp.full((16,), att2[0, 0], _F32)
    parts, ex2q = _sc2a(xl2.reshape(N), xr2.reshape(N), src, dst, att2v, N, E)
    out2, dtot = _tc2b(parts, bias2.reshape(1, 1), N)
    a2 = _sc2b(dtot.reshape(N), dst, ex2q, N, E)
    return out2, edge_index, a2.reshape(E, 1)


# speedup vs baseline: 11.4821x; 1.6612x over previous
import functools

import jax
import jax.numpy as jnp
from jax import lax
from jax.experimental import pallas as pl
from jax.experimental.pallas import tpu as pltpu
from jax.experimental.pallas import tpu_sc as plsc

NS = 16
NC = 2
NW = NC * NS
BK = 16
SLOPE = 0.2
EPS = 1e-16

_F32 = jnp.float32
_I32 = jnp.int32

_SC_PARAMS = pltpu.CompilerParams(use_tc_tiling_on_sc=False,
                                  needs_layout_passes=False)


def _lanes():
    return lax.iota(_I32, 16)


def _splat_i(v):
    return jnp.zeros((16,), _I32) + v


def _zero16():
    return jnp.zeros((16,), _F32)


def _pack_rows(v):
    u = jax.lax.bitcast_convert_type(v, jnp.uint32)
    r = (u + jnp.uint32(0x7FFF) + ((u >> 16) & jnp.uint32(1))) >> 16
    half = v.shape[1] // 2
    w = r[:, :half] | (r[:, half:] << 16)
    return jax.lax.bitcast_convert_type(w, _I32)


def _tc1_body(x_ref, wl_ref, bl_ref, wr_ref, br_ref, xl_ref, xr_ref, xlc_ref,
              xlp_ref, xrp_ref):
    x = x_ref[...]
    xl = jnp.dot(x, wl_ref[...], preferred_element_type=_F32,
                 precision=lax.Precision.HIGHEST) + bl_ref[...]
    xr = jnp.dot(x, wr_ref[...], preferred_element_type=_F32,
                 precision=lax.Precision.HIGHEST) + br_ref[...]
    xl_ref[...] = xl
    xr_ref[...] = xr
    for c in range(4):
        xlc_ref[c] = xl[:, c * 128:(c + 1) * 128]
    xlp_ref[...] = _pack_rows(xl)
    xrp_ref[...] = _pack_rows(xr)


def _tc1(x, wl, bl, wr, br, N, D, HF):
    BN = 1000
    return pl.pallas_call(
        _tc1_body,
        grid=(N // BN,),
        in_specs=[
            pl.BlockSpec((BN, D), lambda i: (i, 0)),
            pl.BlockSpec((D, HF), lambda i: (0, 0)),
            pl.BlockSpec((1, HF), lambda i: (0, 0)),
            pl.BlockSpec((D, HF), lambda i: (0, 0)),
            pl.BlockSpec((1, HF), lambda i: (0, 0)),
        ],
        out_specs=[
            pl.BlockSpec((BN, HF), lambda i: (i, 0)),
            pl.BlockSpec((BN, HF), lambda i: (i, 0)),
            pl.BlockSpec((4, BN, 128), lambda i: (0, i, 0)),
            pl.BlockSpec((BN, HF // 2), lambda i: (i, 0)),
            pl.BlockSpec((BN, HF // 2), lambda i: (i, 0)),
        ],
        out_shape=[
            jax.ShapeDtypeStruct((N, HF), _F32),
            jax.ShapeDtypeStruct((N, HF), _F32),
            jax.ShapeDtypeStruct((4, N, 128), _F32),
            jax.ShapeDtypeStruct((N, HF // 2), _I32),
            jax.ShapeDtypeStruct((N, HF // 2), _I32),
        ],
    )(x, wl, bl, wr, br)


def _sc1a_body(N, E, xl_hbm, xr_hbm, src_hbm, dst_hbm, att_hbm,
               den_hbm, exq_hbm,
               accum, src_slab, dst_slab, xl_rows2, xr_rows2,
               ex_stage2, att_buf, zero_buf, sem_xl, sem_xr, sem_ex, sem_sca):
    c = lax.axis_index("c")
    s = lax.axis_index("s")
    wid = c * NS + s
    EW = E // NW
    NB = EW // BK
    RS = N // NS
    lanes = _lanes()

    pltpu.sync_copy(att_hbm, att_buf)
    pltpu.sync_copy(src_hbm.at[pl.ds(wid * EW, EW)], src_slab)
    pltpu.sync_copy(dst_hbm.at[pl.ds(wid * EW, EW)], dst_slab)

    @pl.loop(0, 25)
    def _(i):
        zero_buf[i, pl.ds(0, 16)] = _zero16()

    @pl.loop(0, BK)
    def _(e):
        for p in range(2):
            ex_stage2[p, e, pl.ds(0, 16)] = _zero16()

    @pl.loop(0, RS // 25)
    def _(k):
        pltpu.sync_copy(zero_buf, accum.at[pl.ds(s * RS + k * 25, 25), :])

    plsc.subcore_barrier()

    for q in range(3):
        srcv0 = src_slab[pl.ds(q * BK, BK)]
        dstv0 = dst_slab[pl.ds(q * BK, BK)]
        pltpu.async_copy(xl_hbm.at[srcv0], xl_rows2.at[q], sem_xl.at[q])
        pltpu.async_copy(xr_hbm.at[dstv0], xr_rows2.at[q], sem_xr.at[q])

    @pl.loop(0, NB)
    def _(g):
        p = g & 3
        base = wid * EW + g * BK
        srcv = src_slab[pl.ds(g * BK, BK)]
        dstv = dst_slab[pl.ds(g * BK, BK)]
        pltpu.make_async_copy(xl_hbm.at[srcv], xl_rows2.at[p], sem_xl.at[p]).wait()
        pltpu.make_async_copy(xr_hbm.at[dstv], xr_rows2.at[p], sem_xr.at[p]).wait()

        @pl.when(g < NB - 3)
        def _():
            pf = (g + 3) & 3
            srcv2 = src_slab[pl.ds((g + 3) * BK, BK)]
            dstv2 = dst_slab[pl.ds((g + 3) * BK, BK)]
            pltpu.async_copy(xl_hbm.at[srcv2], xl_rows2.at[pf], sem_xl.at[pf])
            pltpu.async_copy(xr_hbm.at[dstv2], xr_rows2.at[pf], sem_xr.at[pf])

        pe = g & 1

        @pl.when(g >= 2)
        def _():
            pltpu.make_async_copy(ex_stage2.at[pe], exq_hbm.at[pl.ds(base, BK)],
                                  sem_ex.at[pe]).wait()
            pltpu.make_async_copy(ex_stage2.at[pe], accum.at[dstv],
                                  sem_sca.at[pe]).wait()

        splat_p = _splat_i(0) + p
        for hp in range(4):
            def fbody(jo, accs, hp=hp):
                acc_lo, acc_hi = accs
                attl = att_buf[hp, pl.ds(jo * 16, 16)]
                atth = att_buf[hp + 4, pl.ds(jo * 16, 16)]
                for jj in range(16):
                    col = _splat_i(hp * 64 + jo * 16 + jj)
                    wl = plsc.load_gather(xl_rows2, [splat_p, lanes, col])
                    wr = plsc.load_gather(xr_rows2, [splat_p, lanes, col])
                    xl_lo, xl_hi = plsc.unpack(
                        plsc.bitcast(wl, jnp.bfloat16),
                        format=plsc.PackFormat.INTERLEAVED)
                    xr_lo, xr_hi = plsc.unpack(
                        plsc.bitcast(wr, jnp.bfloat16),
                        format=plsc.PackFormat.INTERLEAVED)
                    m_lo = xl_lo + xr_lo
                    m_hi = xl_hi + xr_hi
                    lk_lo = jnp.maximum(m_lo, SLOPE * m_lo)
                    lk_hi = jnp.maximum(m_hi, SLOPE * m_hi)
                    acc_lo = acc_lo + attl[jj] * lk_lo
                    acc_hi = acc_hi + atth[jj] * lk_hi
                return (acc_lo, acc_hi)
            acc_lo, acc_hi = lax.fori_loop(0, 4, fbody,
                                           (_zero16(), _zero16()))
            for h, a in ((hp, acc_lo), (hp + 4, acc_hi)):
                exh = jnp.exp(a)
                plsc.store_scatter(ex_stage2, [_splat_i(0) + pe, lanes,
                                               _splat_i(h)], exh)

        pltpu.async_copy(ex_stage2.at[pe], accum.at[dstv], sem_sca.at[pe],
                         add=True)
        pltpu.async_copy(ex_stage2.at[pe], exq_hbm.at[pl.ds(base, BK)],
                         sem_ex.at[pe])

    dstv_last = dst_slab[pl.ds((NB - 1) * BK, BK)]
    for p in range(2):
        pltpu.make_async_copy(ex_stage2.at[p], exq_hbm.at[pl.ds(wid * EW, BK)],
                              sem_ex.at[p]).wait()
        pltpu.make_async_copy(ex_stage2.at[p], accum.at[dstv_last],
                              sem_sca.at[p]).wait()

    plsc.subcore_barrier()
    pltpu.sync_copy(accum.at[pl.ds(s * RS, RS), :],
                    den_hbm.at[c, pl.ds(s * RS, RS)])


def _sc1a(xl, xr, src, dst, att, N, E):
    EW = E // NW
    mesh = plsc.VectorSubcoreMesh(core_axis_name="c", subcore_axis_name="s")
    return pl.kernel(
        functools.partial(_sc1a_body, N, E),
        out_type=(
            jax.ShapeDtypeStruct((NC, N, 16), _F32),
            jax.ShapeDtypeStruct((E, 16), _F32),
        ),
        mesh=mesh,
        compiler_params=_SC_PARAMS,
        scratch_types=[
            pltpu.VMEM_SHARED((N, 16), _F32),
            pltpu.VMEM((EW,), _I32),
            pltpu.VMEM((EW,), _I32),
            pltpu.VMEM((4, BK, 256), _I32),
            pltpu.VMEM((4, BK, 256), _I32),
            pltpu.VMEM((2, BK, 16), _F32),
            pltpu.VMEM((8, 64), _F32),
            pltpu.VMEM((25, 16), _F32),
            pltpu.SemaphoreType.DMA((4,)),
            pltpu.SemaphoreType.DMA((4,)),
            pltpu.SemaphoreType.DMA((2,)),
            pltpu.SemaphoreType.DMA((2,)),
        ],
    )(xl, xr, src, dst, att)


def _sc1b_body(N, E, xlc_hbm, src_hbm, dst_hbm, exq_hbm,
               un_hbm,
               accum, src_slab, dst_slab, xc_rows2, ex_stage2, row_buf2,
               zero_buf, sem_xc, sem_ex, sem_sc):
    c = lax.axis_index("c")
    s = lax.axis_index("s")
    wid = c * NS + s
    EW = E // NW
    NB = EW // BK
    RS = N // NS
    lanes = _lanes()

    pltpu.sync_copy(src_hbm.at[pl.ds(wid * EW, EW)], src_slab)
    pltpu.sync_copy(dst_hbm.at[pl.ds(wid * EW, EW)], dst_slab)

    @pl.loop(0, 25)
    def _(i):
        for j in range(8):
            zero_buf[i, pl.ds(j * 16, 16)] = _zero16()

    def zero_accum():
        @pl.loop(0, RS // 25)
        def _(k):
            pltpu.sync_copy(zero_buf, accum.at[pl.ds(s * RS + k * 25, 25), :])

    zero_accum()
    plsc.subcore_barrier()

    for ch in range(4):
        for q in range(3):
            srcv0 = src_slab[pl.ds(q * BK, BK)]
            pltpu.async_copy(xlc_hbm.at[ch].at[srcv0], xc_rows2.at[q],
                             sem_xc.at[q])
            pltpu.async_copy(exq_hbm.at[pl.ds(wid * EW + q * BK, BK)],
                             ex_stage2.at[q], sem_ex.at[q])

        @pl.loop(0, NB)
        def _(g, ch=ch):
            p = g & 3
            pe = g & 1
            base = wid * EW + g * BK
            srcv = src_slab[pl.ds(g * BK, BK)]
            dstv = dst_slab[pl.ds(g * BK, BK)]
            pltpu.make_async_copy(xlc_hbm.at[ch].at[srcv], xc_rows2.at[p],
                                  sem_xc.at[p]).wait()
            pltpu.make_async_copy(exq_hbm.at[pl.ds(base, BK)], ex_stage2.at[p],
                                  sem_ex.at[p]).wait()

            @pl.when(g >= 2)
            def _():
                pltpu.make_async_copy(row_buf2.at[pe], accum.at[dstv],
                                      sem_sc.at[pe]).wait()

            @pl.when(g < NB - 3)
            def _():
                pf = (g + 3) & 3
                srcv2 = src_slab[pl.ds((g + 3) * BK, BK)]
                pltpu.async_copy(xlc_hbm.at[ch].at[srcv2], xc_rows2.at[pf],
                                 sem_xc.at[pf])
                pltpu.async_copy(exq_hbm.at[pl.ds(base + 3 * BK, BK)],
                                 ex_stage2.at[pf], sem_ex.at[pf])

            @pl.loop(0, BK)
            def _(e, ch=ch):
                exv = ex_stage2[p, e, pl.ds(0, 16)]
                for j in range(8):
                    row_buf2[pe, e, pl.ds(j * 16, 16)] = (
                        xc_rows2[p, e, pl.ds(j * 16, 16)] * exv[2 * ch + j // 4])

            pltpu.async_copy(row_buf2.at[pe], accum.at[dstv], sem_sc.at[pe],
                             add=True)

        dstv_last = dst_slab[pl.ds((NB - 1) * BK, BK)]
        for p in range(2):
            pltpu.make_async_copy(row_buf2.at[p], accum.at[dstv_last],
                                  sem_sc.at[p]).wait()
        plsc.subcore_barrier()
        pltpu.sync_copy(accum.at[pl.ds(s * RS, RS), :],
                        un_hbm.at[c, ch, pl.ds(s * RS, RS)])
        if ch < 3:
            zero_accum()
        plsc.subcore_barrier()


def _sc1b(xlc, src, dst, exq, N, E):
    EW = E // NW
    mesh = plsc.VectorSubcoreMesh(core_axis_name="c", subcore_axis_name="s")
    return pl.kernel(
        functools.partial(_sc1b_body, N, E),
        out_type=jax.ShapeDtypeStruct((NC, 4, N, 128), _F32),
        mesh=mesh,
        compiler_params=_SC_PARAMS,
        scratch_types=[
            pltpu.VMEM_SHARED((N, 128), _F32),
            pltpu.VMEM((EW,), _I32),
            pltpu.VMEM((EW,), _I32),
            pltpu.VMEM((4, BK, 128), _F32),
            pltpu.VMEM((4, BK, 16), _F32),
            pltpu.VMEM((2, BK, 128), _F32),
            pltpu.VMEM((25, 128), _F32),
            pltpu.SemaphoreType.DMA((4,)),
            pltpu.SemaphoreType.DMA((4,)),
            pltpu.SemaphoreType.DMA((2,)),
        ],
    )(xlc, src, dst, exq)


def _tc2_body(u_ref, d_ref, b1_ref, wl2_ref, bl2_ref, wr2_ref, br2_ref,
              xl2_ref, xr2_ref):
    u = u_ref[...]
    us = u[0] + u[1]
    d = d_ref[...]
    dsum = d[0] + d[1]
    cols = []
    for ch in range(4):
        for k in range(2):
            dh = dsum[:, 2 * ch + k:2 * ch + k + 1] + EPS
            cols.append(us[ch][:, 64 * k:64 * (k + 1)] / dh)
    h = jnp.concatenate(cols, axis=1) + b1_ref[...]
    h = jnp.maximum(h, 0.0)
    xl2_ref[...] = jnp.dot(h, wl2_ref[...], preferred_element_type=_F32,
                           precision=lax.Precision.HIGHEST) + bl2_ref[...]
    xr2_ref[...] = jnp.dot(h, wr2_ref[...], preferred_element_type=_F32,
                           precision=lax.Precision.HIGHEST) + br2_ref[...]


def _tc2(un, den, b1, wl2, bl2, wr2, br2, N, HF):
    BN = 1000
    return pl.pallas_call(
        _tc2_body,
        grid=(N // BN,),
        in_specs=[
            pl.BlockSpec((NC, 4, BN, 128), lambda i: (0, 0, i, 0)),
            pl.BlockSpec((NC, BN, 16), lambda i: (0, i, 0)),
            pl.BlockSpec((1, HF), lambda i: (0, 0)),
            pl.BlockSpec((HF, 1), lambda i: (0, 0)),
            pl.BlockSpec((1, 1), lambda i: (0, 0)),
            pl.BlockSpec((HF, 1), lambda i: (0, 0)),
            pl.BlockSpec((1, 1), lambda i: (0, 0)),
        ],
        out_specs=[
            pl.BlockSpec((BN, 1), lambda i: (i, 0)),
            pl.BlockSpec((BN, 1), lambda i: (i, 0)),
        ],
        out_shape=[
            jax.ShapeDtypeStruct((N, 1), _F32),
            jax.ShapeDtypeStruct((N, 1), _F32),
        ],
    )(un, den, b1, wl2, bl2, wr2, br2)


def _sc2a_body(N, E, xl2_hbm, xr2_hbm, src_hbm, dst_hbm, att2_hbm,
               parts_hbm, ex2q_hbm,
               accum2, xl2_buf, xr2_buf, att2_buf, src_slab, dst_slab,
               ex2_slab, row2_buf, zero2_buf):
    c = lax.axis_index("c")
    s = lax.axis_index("s")
    wid = c * NS + s
    EW = E // NW
    NB = EW // BK
    RS = N // NS
    ZR = 125
    lanes = _lanes()

    pltpu.sync_copy(xl2_hbm, xl2_buf)
    pltpu.sync_copy(xr2_hbm, xr2_buf)
    pltpu.sync_copy(att2_hbm, att2_buf)
    pltpu.sync_copy(src_hbm.at[pl.ds(wid * EW, EW)], src_slab)
    pltpu.sync_copy(dst_hbm.at[pl.ds(wid * EW, EW)], dst_slab)

    @pl.loop(0, ZR)
    def _(i):
        zero2_buf[i, pl.ds(0, 16)] = _zero16()

    @pl.loop(0, BK)
    def _(e):
        row2_buf[e, pl.ds(0, 16)] = _zero16()

    @pl.loop(0, RS // ZR)
    def _(k):
        pltpu.sync_copy(zero2_buf, accum2.at[pl.ds(s * RS + k * ZR, ZR), :])

    plsc.subcore_barrier()
    attv = att2_buf[...]

    @pl.loop(0, NB)
    def _(b):
        srcv = src_slab[pl.ds(b * BK, BK)]
        dstv = dst_slab[pl.ds(b * BK, BK)]
        xls = plsc.load_gather(xl2_buf, [srcv])
        xrd = plsc.load_gather(xr2_buf, [dstv])
        m = xls + xrd
        lk = jnp.maximum(m, SLOPE * m)
        ex2 = jnp.exp(attv * lk)
        plsc.store_scatter(row2_buf, [lanes, _splat_i(0)], ex2 * xls)
        plsc.store_scatter(row2_buf, [lanes, _splat_i(1)], ex2)
        ex2_slab[pl.ds(b * BK, BK)] = ex2
        pltpu.sync_copy(row2_buf, accum2.at[dstv], add=True)

    pltpu.sync_copy(ex2_slab, ex2q_hbm.at[pl.ds(wid * EW, EW)])
    plsc.subcore_barrier()
    pltpu.sync_copy(accum2.at[pl.ds(s * RS, RS), :],
                    parts_hbm.at[c, pl.ds(s * RS, RS)])


def _sc2a(xl2, xr2, src, dst, att2v, N, E):
    EW = E // NW
    mesh = plsc.VectorSubcoreMesh(core_axis_name="c", subcore_axis_name="s")
    return pl.kernel(
        functools.partial(_sc2a_body, N, E),
        out_type=(
            jax.ShapeDtypeStruct((NC, N, 16), _F32),
            jax.ShapeDtypeStruct((E,), _F32),
        ),
        mesh=mesh,
        compiler_params=_SC_PARAMS,
        scratch_types=[
            pltpu.VMEM_SHARED((N, 16), _F32),
            pltpu.VMEM((N,), _F32),
            pltpu.VMEM((N,), _F32),
            pltpu.VMEM((16,), _F32),
            pltpu.VMEM((EW,), _I32),
            pltpu.VMEM((EW,), _I32),
            pltpu.VMEM((EW,), _F32),
            pltpu.VMEM((BK, 16), _F32),
            pltpu.VMEM((125, 16), _F32),
        ],
    )(xl2, xr2, src, dst, att2v)


def _tc2b_body(p_ref, b2_ref, out2_ref, dtot_ref):
    p = p_ref[...]
    u = p[0, :, 0:1] + p[1, :, 0:1]
    dd = p[0, :, 1:2] + p[1, :, 1:2] + EPS
    out2_ref[...] = u / dd + b2_ref[...]
    dtot_ref[...] = dd


def _tc2b(parts, b2, N):
    BN = 1000
    return pl.pallas_call(
        _tc2b_body,
        grid=(N // BN,),
        in_specs=[
            pl.BlockSpec((NC, BN, 16), lambda i: (0, i, 0)),
            pl.BlockSpec((1, 1), lambda i: (0, 0)),
        ],
        out_specs=[
            pl.BlockSpec((BN, 1), lambda i: (i, 0)),
            pl.BlockSpec((BN, 1), lambda i: (i, 0)),
        ],
        out_shape=[
            jax.ShapeDtypeStruct((N, 1), _F32),
            jax.ShapeDtypeStruct((N, 1), _F32),
        ],
    )(parts, b2)


def _sc2b_body(N, E, dtot_hbm, dst_hbm, ex2q_hbm, a2_hbm,
               dt_buf, dst_slab, e2_slab, a2_slab):
    c = lax.axis_index("c")
    s = lax.axis_index("s")
    wid = c * NS + s
    EW = E // NW
    NB = EW // BK

    pltpu.sync_copy(dtot_hbm, dt_buf)
    pltpu.sync_copy(dst_hbm.at[pl.ds(wid * EW, EW)], dst_slab)
    pltpu.sync_copy(ex2q_hbm.at[pl.ds(wid * EW, EW)], e2_slab)

    @pl.loop(0, NB)
    def _(b):
        dstv = dst_slab[pl.ds(b * BK, BK)]
        dv = plsc.load_gather(dt_buf, [dstv])
        a2_slab[pl.ds(b * BK, BK)] = e2_slab[pl.ds(b * BK, BK)] / dv

    pltpu.sync_copy(a2_slab, a2_hbm.at[pl.ds(wid * EW, EW)])


def _sc2b(dtot, dst, ex2q, N, E):
    EW = E // NW
    mesh = plsc.VectorSubcoreMesh(core_axis_name="c", subcore_axis_name="s")
    return pl.kernel(
        functools.partial(_sc2b_body, N, E),
        out_type=jax.ShapeDtypeStruct((E,), _F32),
        mesh=mesh,
        compiler_params=_SC_PARAMS,
        scratch_types=[
            pltpu.VMEM((N,), _F32),
            pltpu.VMEM((EW,), _I32),
            pltpu.VMEM((EW,), _F32),
            pltpu.VMEM((EW,), _F32),
        ],
    )(dtot, dst, ex2q)


def kernel(x, edge_index, Wl1, bl1, Wr1, br1, att1, bias1,
           Wl2, bl2, Wr2, br2, att2, bias2):
    N, D = x.shape
    E = edge_index.shape[1]
    HF = Wl1.shape[1]
    src = edge_index[0]
    dst = edge_index[1]

    xl, xr, xlc, xlp, xrp = _tc1(x, Wl1, bl1.reshape(1, HF),
                                 Wr1, br1.reshape(1, HF), N, D, HF)
    den, exq = _sc1a(xlp, xrp, src, dst, att1, N, E)
    un = _sc1b(xlc, src, dst, exq, N, E)
    xl2, xr2 = _tc2(un, den, bias1.reshape(1, HF), Wl2, bl2.reshape(1, 1),
                    Wr2, br2.reshape(1, 1), N, HF)
    att2v = jnp.full((16,), att2[0, 0], _F32)
    parts, ex2q = _sc2a(xl2.reshape(N), xr2.reshape(N), src, dst, att2v, N, E)
    out2, dtot = _tc2b(parts, bias2.reshape(1, 1), N)
    a2 = _sc2b(dtot.reshape(N), dst, ex2q, N, E)
    return out2, edge_index, a2.reshape(E, 1)
